# baseline (speedup 1.0000x reference)
import jax
import jax.numpy as jnp
from jax import lax
from jax.experimental import pallas as pl
from jax.experimental.pallas import tpu as pltpu
from jax.experimental.pallas import tpu_sc as plsc

N = 10000
E = 320000
NUM_CLASSES = 10
NPAD = 10240
NC, NS = 2, 16
CHUNK = 128
NCHUNKS = E // CHUNK
RPT = NPAD // NS
BLK = 10240
GRID = NPAD // BLK

_MESH = plsc.VectorSubcoreMesh(core_axis_name="c", subcore_axis_name="s")


DEG_IB = 6
NW = NC * NS


def _deg_kernel(ei_hbm, out_hbm, ib, ones_v, zeros_v, acc_sh, isem):
    cid = lax.axis_index("c")
    sid = lax.axis_index("s")
    w = sid * NC + cid

    def idx_start(c, b):
        pltpu.make_async_copy(ei_hbm.at[1, pl.ds(c * CHUNK, CHUNK)], ib.at[b],
                              isem).start()

    def idx_wait(b):
        pltpu.make_async_copy(ei_hbm.at[1, pl.ds(0, CHUNK)], ib.at[b],
                              isem).wait()

    def s_sync(b):
        pltpu.sync_copy(ones_v, acc_sh.at[ib.at[b]], add=True)

    for j in range(CHUNK // 16):
        ones_v[pl.ds(j * 16, 16)] = jnp.ones((16,), jnp.float32)
    for j in range(RPT // 16):
        zeros_v[pl.ds(j * 16, 16)] = jnp.zeros((16,), jnp.float32)
    pltpu.sync_copy(zeros_v, acc_sh.at[pl.ds(sid * RPT, RPT)])
    for b in range(DEG_IB):
        idx_start(w + b * NW, b)
    plsc.subcore_barrier()

    def group(j, carry):
        for b in range(DEG_IB):
            k = j * DEG_IB + b
            idx_wait(b)
            s_sync(b)

            @pl.when(k + DEG_IB < NCHUNKS // NW)
            def _():
                idx_start(w + (k + DEG_IB) * NW, b)
        return carry

    lax.fori_loop(0, (NCHUNKS // NW) // DEG_IB, group, 0)

    @pl.when(w < NCHUNKS % NW)
    def _():
        idx_start(NCHUNKS - NCHUNKS % NW + w, 0)
        idx_wait(0)
        s_sync(0)

    plsc.subcore_barrier()
    pltpu.sync_copy(acc_sh.at[pl.ds(sid * RPT, RPT)],
                    out_hbm.at[pl.ds(cid * NPAD + sid * RPT, RPT)])


def _degrees(ei):
    k = pl.kernel(
        _deg_kernel,
        out_type=jax.ShapeDtypeStruct((2 * NPAD,), jnp.float32),
        mesh=_MESH,
        scratch_types=[
            pltpu.VMEM((DEG_IB, CHUNK), jnp.int32),
            pltpu.VMEM((CHUNK,), jnp.float32),
            pltpu.VMEM((RPT,), jnp.float32),
            pltpu.VMEM_SHARED((NPAD,), jnp.float32),
            pltpu.SemaphoreType.DMA,
        ],
        compiler_params=pltpu.CompilerParams(use_tc_tiling_on_sc=True),
    )
    return k(ei).reshape(2, NPAD)


NFULL = NCHUNKS // NS
NTAIL = NCHUNKS % NS


def _make_prop_body(ibn, rbn):
    def body(hs_hbm, src_hbm, dst_hbm, out_hbm, ib, rb, acc_sh, isem, gsem):
        cid = lax.axis_index("c")
        sid = lax.axis_index("s")

        def idx_start(c, b):
            pltpu.make_async_copy(src_hbm.at[c], ib.at[b, 0], isem).start()
            pltpu.make_async_copy(dst_hbm.at[c], ib.at[b, 1], isem).start()

        def idx_wait(b):
            pltpu.make_async_copy(src_hbm.at[0], ib.at[b, 0], isem).wait()
            pltpu.make_async_copy(dst_hbm.at[0], ib.at[b, 1], isem).wait()

        def g_start(kb, rbuf):
            pltpu.make_async_copy(hs_hbm.at[cid].at[ib.at[kb, 0]], rb.at[rbuf],
                                  gsem).start()

        def g_wait(kb, rbuf):
            pltpu.make_async_copy(hs_hbm.at[cid].at[ib.at[kb, 0]], rb.at[rbuf],
                                  gsem).wait()

        def s_sync(kb, rbuf):
            pltpu.sync_copy(rb.at[rbuf], acc_sh.at[ib.at[kb, 1]], add=True)

        pltpu.sync_copy(hs_hbm.at[cid, pl.ds(sid * RPT, RPT)],
                        acc_sh.at[pl.ds(sid * RPT, RPT)])
        for b in range(ibn):
            idx_start(sid + b * NS, b)
        plsc.subcore_barrier()
        for b in range(rbn):
            idx_wait(b)
            g_start(b, b)

        def group(j, carry):
            for b in range(ibn):
                k = j * ibn + b
                g_wait(b, b % rbn)
                s_sync(b, b % rbn)

                @pl.when(k + ibn < NFULL)
                def _():
                    idx_start(sid + (k + ibn) * NS, b)

                @pl.when(k + rbn < NFULL)
                def _(bb=(b + rbn) % ibn):
                    idx_wait(bb)
                    g_start(bb, bb % rbn)
            return carry

        lax.fori_loop(0, NFULL // ibn, group, 0)

        @pl.when(sid < NTAIL)
        def _():
            idx_start(NFULL * NS + sid, 0)
            idx_wait(0)
            g_start(0, 0)
            g_wait(0, 0)
            s_sync(0, 0)

        plsc.subcore_barrier()
        pltpu.sync_copy(acc_sh.at[pl.ds(sid * RPT, RPT)],
                        out_hbm.at[cid, pl.ds(sid * RPT, RPT)])

    return body


NHALF = NCHUNKS // NC
NFULL_B = NHALF // NS
NTAIL_B = NHALF % NS
ZR = 64


def _make_prop_full_body(ibn, rbn):
    def body(hs_hbm, ei_hbm, out_hbm, ib, rb, zbuf, acc_sh,
             isem, gsem):
        cid = lax.axis_index("c")
        sid = lax.axis_index("s")
        c0 = cid * NHALF

        def idx_start(c, b):
            pltpu.make_async_copy(ei_hbm.at[0, pl.ds(c * CHUNK, CHUNK)],
                                  ib.at[b, 0], isem).start()
            pltpu.make_async_copy(ei_hbm.at[1, pl.ds(c * CHUNK, CHUNK)],
                                  ib.at[b, 1], isem).start()

        def idx_wait(b):
            pltpu.make_async_copy(ei_hbm.at[0, pl.ds(0, CHUNK)],
                                  ib.at[b, 0], isem).wait()
            pltpu.make_async_copy(ei_hbm.at[1, pl.ds(0, CHUNK)],
                                  ib.at[b, 1], isem).wait()

        def g_start(kb, rbuf):
            pltpu.make_async_copy(hs_hbm.at[ib.at[kb, 0]], rb.at[rbuf],
                                  gsem).start()

        def g_wait(kb, rbuf):
            pltpu.make_async_copy(hs_hbm.at[ib.at[kb, 0]], rb.at[rbuf],
                                  gsem).wait()

        def s_sync(kb, rbuf):
            pltpu.sync_copy(rb.at[rbuf], acc_sh.at[ib.at[kb, 1]], add=True)

        for b in range(ibn):
            idx_start(c0 + sid + b * NS, b)

        @pl.when(cid == 0)
        def _():
            pltpu.sync_copy(hs_hbm.at[pl.ds(sid * RPT, RPT)],
                            acc_sh.at[pl.ds(sid * RPT, RPT)])

        @pl.when(cid == 1)
        def _():
            for r in range(ZR):
                for j in range(128 // 16):
                    zbuf[r, pl.ds(j * 16, 16)] = jnp.zeros((16,), jnp.float32)
            for r in range(RPT // ZR):
                pltpu.sync_copy(zbuf,
                                acc_sh.at[pl.ds(sid * RPT + r * ZR, ZR)])

        plsc.subcore_barrier()
        for b in range(rbn):
            idx_wait(b)
            g_start(b, b)

        def group(j, carry):
            for b in range(ibn):
                k = j * ibn + b
                g_wait(b, b % rbn)
                s_sync(b, b % rbn)

                @pl.when(k + ibn < NFULL_B)
                def _():
                    idx_start(c0 + sid + (k + ibn) * NS, b)

                @pl.when(k + rbn < NFULL_B)
                def _(bb=(b + rbn) % ibn):
                    idx_wait(bb)
                    g_start(bb, bb % rbn)
            return carry

        lax.fori_loop(0, NFULL_B // ibn, group, 0)

        @pl.when(sid < NTAIL_B)
        def _():
            idx_start(c0 + NFULL_B * NS + sid, 0)
            idx_wait(0)
            g_start(0, 0)
            g_wait(0, 0)
            s_sync(0, 0)

        plsc.subcore_barrier()
        pltpu.sync_copy(acc_sh.at[pl.ds(sid * RPT, RPT)],
                        out_hbm.at[cid, pl.ds(sid * RPT, RPT)])

    return body


def _propagate_full(hs, ei):
    ibn, rbn = 6, 2
    k = pl.kernel(
        _make_prop_full_body(ibn, rbn),
        out_type=jax.ShapeDtypeStruct((2, NPAD, 128), jnp.float32),
        mesh=_MESH,
        scratch_types=[
            pltpu.VMEM((ibn, 2, CHUNK), jnp.int32),
            pltpu.VMEM((rbn, CHUNK, 128), jnp.float32),
            pltpu.VMEM((ZR, 128), jnp.float32),
            pltpu.VMEM_SHARED((NPAD, 128), jnp.float32),
            pltpu.SemaphoreType.DMA,
            pltpu.SemaphoreType.DMA,
        ],
        compiler_params=pltpu.CompilerParams(use_tc_tiling_on_sc=True),
    )
    return k(hs, ei)


def _propagate(hs, src2d, dst2d, dc):
    ibn, rbn = (4, 2) if dc == 128 else (12, 6)
    k = pl.kernel(
        _make_prop_body(ibn, rbn),
        out_type=jax.ShapeDtypeStruct((2, NPAD, dc), jnp.float32),
        mesh=_MESH,
        scratch_types=[
            pltpu.VMEM((ibn, 2, CHUNK), jnp.int32),
            pltpu.VMEM((rbn, CHUNK, dc), jnp.float32),
            pltpu.VMEM_SHARED((NPAD, dc), jnp.float32),
            pltpu.SemaphoreType.DMA,
            pltpu.SemaphoreType.DMA,
        ],
        compiler_params=pltpu.CompilerParams(use_tc_tiling_on_sc=(dc == 128)),
    )
    return k(hs, src2d, dst2d)


def _tc_first(xp, W, degp_t):
    dout = W.shape[1]

    def body(x_ref, w_ref, deg_ref, hs_ref, dis_ref):
        deg = 1.0 + deg_ref[:, 0:1] + deg_ref[:, 1:2]
        d = lax.rsqrt(deg)
        dis_ref[...] = d
        h = jnp.dot(x_ref[...], w_ref[...],
                    preferred_element_type=jnp.float32)
        hs_ref[...] = h * d

    return pl.pallas_call(
        body,
        grid=(GRID,),
        in_specs=[
            pl.BlockSpec((BLK, xp.shape[1]), lambda i: (i, 0)),
            pl.BlockSpec(W.shape, lambda i: (0, 0)),
            pl.BlockSpec((BLK, 2), lambda i: (i, 0)),
        ],
        out_specs=[
            pl.BlockSpec((BLK, dout), lambda i: (i, 0)),
            pl.BlockSpec((BLK, 1), lambda i: (i, 0)),
        ],
        out_shape=[
            jax.ShapeDtypeStruct((NPAD, dout), jnp.float32),
            jax.ShapeDtypeStruct((NPAD, 1), jnp.float32),
        ],
    )(xp, W, degp_t)


def _tc_mid(S, dis, b, W, sum_in, split_out):
    din, dout = W.shape

    def body(s_ref, dis_ref, b_ref, w_ref, out_ref):
        if sum_in:
            X = s_ref[0] + s_ref[1]
        else:
            X = jnp.concatenate([s_ref[0], s_ref[1]], axis=-1)
        d = dis_ref[...]
        X = jnp.maximum(X * d + b_ref[...], 0.0)
        h = jnp.dot(X, w_ref[...],
                    preferred_element_type=jnp.float32)
        hs = h * d
        if split_out:
            out_ref[0] = hs[:, :dout // 2]
            out_ref[1] = hs[:, dout // 2:]
        else:
            out_ref[...] = hs

    s_half = S.shape[2]
    if split_out:
        out_spec = pl.BlockSpec((2, BLK, dout // 2), lambda i: (0, i, 0))
        out_shape = jax.ShapeDtypeStruct((2, NPAD, dout // 2), jnp.float32)
    else:
        out_spec = pl.BlockSpec((BLK, dout), lambda i: (i, 0))
        out_shape = jax.ShapeDtypeStruct((NPAD, dout), jnp.float32)
    return pl.pallas_call(
        body,
        grid=(GRID,),
        in_specs=[
            pl.BlockSpec((2, BLK, s_half), lambda i: (0, i, 0)),
            pl.BlockSpec((BLK, 1), lambda i: (i, 0)),
            pl.BlockSpec((1, din), lambda i: (0, 0)),
            pl.BlockSpec(W.shape, lambda i: (0, 0)),
        ],
        out_specs=out_spec,
        out_shape=out_shape,
    )(S, dis, b, W)


def _tc_final(S, dis, b):
    dpad = 2 * S.shape[2]

    def body(s_ref, dis_ref, b_ref, out_ref):
        z = jnp.concatenate([s_ref[0], s_ref[1]], axis=-1)
        z = z * dis_ref[...] + b_ref[...]
        col = lax.broadcasted_iota(jnp.int32, z.shape, 1)
        mask = col < NUM_CLASSES
        zm = jnp.where(mask, z, -1e30)
        m = jnp.max(zm, axis=1, keepdims=True)
        e = jnp.where(mask, jnp.exp(z - m), 0.0)
        s = jnp.sum(e, axis=1, keepdims=True)
        out_ref[...] = z - (m + jnp.log(s))

    return pl.pallas_call(
        body,
        grid=(GRID,),
        in_specs=[
            pl.BlockSpec((2, BLK, dpad // 2), lambda i: (0, i, 0)),
            pl.BlockSpec((BLK, 1), lambda i: (i, 0)),
            pl.BlockSpec((1, dpad), lambda i: (0, 0)),
        ],
        out_specs=pl.BlockSpec((BLK, dpad), lambda i: (i, 0)),
        out_shape=jax.ShapeDtypeStruct((NPAD, dpad), jnp.float32),
    )(S, dis, b)


def kernel(x, edge_index, W1, b1, W2, b2, W3, b3, W4, b4):
    src2d = edge_index[0].reshape(NCHUNKS, CHUNK)
    dst2d = edge_index[1].reshape(NCHUNKS, CHUNK)
    xp = jnp.pad(x, ((0, NPAD - N), (0, 0)))
    d4p = 32
    W4p = jnp.pad(W4, ((0, 0), (0, d4p - W4.shape[1])))
    b4p = jnp.pad(b4, (0, d4p - b4.shape[0]))

    degp_t = _degrees(edge_index).T

    hs1, dis = _tc_first(xp, W1, degp_t)
    s1 = _propagate_full(hs1, edge_index)
    hs2 = _tc_mid(s1, dis, b1.reshape(1, -1), W2, True, True)
    s2 = _propagate(hs2, src2d, dst2d, W2.shape[1] // 2)
    hs3 = _tc_mid(s2, dis, b2.reshape(1, -1), W3, False, False)
    s3 = _propagate_full(hs3, edge_index)
    hs4 = _tc_mid(s3, dis, b3.reshape(1, -1), W4p, True, True)
    s4 = _propagate(hs4, src2d, dst2d, d4p // 2)
    out = _tc_final(s4, dis, b4p.reshape(1, -1))
    return out[:N, :NUM_CLASSES]


# speedup vs baseline: 1.0193x; 1.0193x over previous
import jax
import jax.numpy as jnp
from jax import lax
from jax.experimental import pallas as pl
from jax.experimental.pallas import tpu as pltpu
from jax.experimental.pallas import tpu_sc as plsc

N = 10000
E = 320000
NUM_CLASSES = 10
NPAD = 10240
NC, NS = 2, 16
CHUNK = 128
NCHUNKS = E // CHUNK
RPT = NPAD // NS
BLK = 5120
GRID = NPAD // BLK

_MESH = plsc.VectorSubcoreMesh(core_axis_name="c", subcore_axis_name="s")


DEG_IB = 6
NW = NC * NS


def _deg_kernel(ei_hbm, out_hbm, ib, ones_v, zeros_v, acc_sh, isem):
    cid = lax.axis_index("c")
    sid = lax.axis_index("s")
    w = sid * NC + cid

    def idx_start(c, b):
        pltpu.make_async_copy(ei_hbm.at[1, pl.ds(c * CHUNK, CHUNK)], ib.at[b],
                              isem).start()

    def idx_wait(b):
        pltpu.make_async_copy(ei_hbm.at[1, pl.ds(0, CHUNK)], ib.at[b],
                              isem).wait()

    def s_sync(b):
        pltpu.sync_copy(ones_v, acc_sh.at[ib.at[b]], add=True)

    for j in range(CHUNK // 16):
        ones_v[pl.ds(j * 16, 16)] = jnp.ones((16,), jnp.float32)
    for j in range(RPT // 16):
        zeros_v[pl.ds(j * 16, 16)] = jnp.zeros((16,), jnp.float32)
    pltpu.sync_copy(zeros_v, acc_sh.at[pl.ds(sid * RPT, RPT)])
    for b in range(DEG_IB):
        idx_start(w + b * NW, b)
    plsc.subcore_barrier()

    def group(j, carry):
        for b in range(DEG_IB):
            k = j * DEG_IB + b
            idx_wait(b)
            s_sync(b)

            @pl.when(k + DEG_IB < NCHUNKS // NW)
            def _():
                idx_start(w + (k + DEG_IB) * NW, b)
        return carry

    lax.fori_loop(0, (NCHUNKS // NW) // DEG_IB, group, 0)

    @pl.when(w < NCHUNKS % NW)
    def _():
        idx_start(NCHUNKS - NCHUNKS % NW + w, 0)
        idx_wait(0)
        s_sync(0)

    plsc.subcore_barrier()
    pltpu.sync_copy(acc_sh.at[pl.ds(sid * RPT, RPT)],
                    out_hbm.at[pl.ds(cid * NPAD + sid * RPT, RPT)])


def _degrees(ei):
    k = pl.kernel(
        _deg_kernel,
        out_type=jax.ShapeDtypeStruct((2 * NPAD,), jnp.float32),
        mesh=_MESH,
        scratch_types=[
            pltpu.VMEM((DEG_IB, CHUNK), jnp.int32),
            pltpu.VMEM((CHUNK,), jnp.float32),
            pltpu.VMEM((RPT,), jnp.float32),
            pltpu.VMEM_SHARED((NPAD,), jnp.float32),
            pltpu.SemaphoreType.DMA,
        ],
        compiler_params=pltpu.CompilerParams(use_tc_tiling_on_sc=True),
    )
    return k(ei).reshape(2, NPAD)


NFULL = NCHUNKS // NS
NTAIL = NCHUNKS % NS


def _make_prop_body(ibn, rbn):
    def body(hs_hbm, src_hbm, dst_hbm, out_hbm, ib, rb, acc_sh, isem, gsem):
        cid = lax.axis_index("c")
        sid = lax.axis_index("s")

        def idx_start(c, b):
            pltpu.make_async_copy(src_hbm.at[c], ib.at[b, 0], isem).start()
            pltpu.make_async_copy(dst_hbm.at[c], ib.at[b, 1], isem).start()

        def idx_wait(b):
            pltpu.make_async_copy(src_hbm.at[0], ib.at[b, 0], isem).wait()
            pltpu.make_async_copy(dst_hbm.at[0], ib.at[b, 1], isem).wait()

        def g_start(kb, rbuf):
            pltpu.make_async_copy(hs_hbm.at[cid].at[ib.at[kb, 0]], rb.at[rbuf],
                                  gsem).start()

        def g_wait(kb, rbuf):
            pltpu.make_async_copy(hs_hbm.at[cid].at[ib.at[kb, 0]], rb.at[rbuf],
                                  gsem).wait()

        def s_sync(kb, rbuf):
            pltpu.sync_copy(rb.at[rbuf], acc_sh.at[ib.at[kb, 1]], add=True)

        pltpu.sync_copy(hs_hbm.at[cid, pl.ds(sid * RPT, RPT)],
                        acc_sh.at[pl.ds(sid * RPT, RPT)])
        for b in range(ibn):
            idx_start(sid + b * NS, b)
        plsc.subcore_barrier()
        for b in range(rbn):
            idx_wait(b)
            g_start(b, b)

        def group(j, carry):
            for b in range(ibn):
                k = j * ibn + b
                g_wait(b, b % rbn)
                s_sync(b, b % rbn)

                @pl.when(k + ibn < NFULL)
                def _():
                    idx_start(sid + (k + ibn) * NS, b)

                @pl.when(k + rbn < NFULL)
                def _(bb=(b + rbn) % ibn):
                    idx_wait(bb)
                    g_start(bb, bb % rbn)
            return carry

        lax.fori_loop(0, NFULL // ibn, group, 0)

        @pl.when(sid < NTAIL)
        def _():
            idx_start(NFULL * NS + sid, 0)
            idx_wait(0)
            g_start(0, 0)
            g_wait(0, 0)
            s_sync(0, 0)

        plsc.subcore_barrier()
        pltpu.sync_copy(acc_sh.at[pl.ds(sid * RPT, RPT)],
                        out_hbm.at[cid, pl.ds(sid * RPT, RPT)])

    return body


NHALF = NCHUNKS // NC
NFULL_B = NHALF // NS
NTAIL_B = NHALF % NS
ZR = 64


def _make_prop_full_body(ibn, rbn):
    def body(hs_hbm, ei_hbm, out_hbm, ib, rb, zbuf, acc_sh,
             isem, gsem):
        cid = lax.axis_index("c")
        sid = lax.axis_index("s")
        c0 = cid * NHALF

        def idx_start(c, b):
            pltpu.make_async_copy(ei_hbm.at[0, pl.ds(c * CHUNK, CHUNK)],
                                  ib.at[b, 0], isem).start()
            pltpu.make_async_copy(ei_hbm.at[1, pl.ds(c * CHUNK, CHUNK)],
                                  ib.at[b, 1], isem).start()

        def idx_wait(b):
            pltpu.make_async_copy(ei_hbm.at[0, pl.ds(0, CHUNK)],
                                  ib.at[b, 0], isem).wait()
            pltpu.make_async_copy(ei_hbm.at[1, pl.ds(0, CHUNK)],
                                  ib.at[b, 1], isem).wait()

        def g_start(kb, rbuf):
            pltpu.make_async_copy(hs_hbm.at[ib.at[kb, 0]], rb.at[rbuf],
                                  gsem).start()

        def g_wait(kb, rbuf):
            pltpu.make_async_copy(hs_hbm.at[ib.at[kb, 0]], rb.at[rbuf],
                                  gsem).wait()

        def s_sync(kb, rbuf):
            pltpu.sync_copy(rb.at[rbuf], acc_sh.at[ib.at[kb, 1]], add=True)

        for b in range(ibn):
            idx_start(c0 + sid + b * NS, b)

        @pl.when(cid == 0)
        def _():
            pltpu.sync_copy(hs_hbm.at[pl.ds(sid * RPT, RPT)],
                            acc_sh.at[pl.ds(sid * RPT, RPT)])

        @pl.when(cid == 1)
        def _():
            for r in range(ZR):
                for j in range(128 // 16):
                    zbuf[r, pl.ds(j * 16, 16)] = jnp.zeros((16,), jnp.float32)
            for r in range(RPT // ZR):
                pltpu.sync_copy(zbuf,
                                acc_sh.at[pl.ds(sid * RPT + r * ZR, ZR)])

        plsc.subcore_barrier()
        for b in range(rbn):
            idx_wait(b)
            g_start(b, b)

        def group(j, carry):
            for b in range(ibn):
                k = j * ibn + b
                g_wait(b, b % rbn)
                s_sync(b, b % rbn)

                @pl.when(k + ibn < NFULL_B)
                def _():
                    idx_start(c0 + sid + (k + ibn) * NS, b)

                @pl.when(k + rbn < NFULL_B)
                def _(bb=(b + rbn) % ibn):
                    idx_wait(bb)
                    g_start(bb, bb % rbn)
            return carry

        lax.fori_loop(0, NFULL_B // ibn, group, 0)

        @pl.when(sid < NTAIL_B)
        def _():
            idx_start(c0 + NFULL_B * NS + sid, 0)
            idx_wait(0)
            g_start(0, 0)
            g_wait(0, 0)
            s_sync(0, 0)

        plsc.subcore_barrier()
        pltpu.sync_copy(acc_sh.at[pl.ds(sid * RPT, RPT)],
                        out_hbm.at[cid, pl.ds(sid * RPT, RPT)])

    return body


def _propagate_full(hs, ei):
    ibn, rbn = 6, 2
    k = pl.kernel(
        _make_prop_full_body(ibn, rbn),
        out_type=jax.ShapeDtypeStruct((2, NPAD, 128), jnp.float32),
        mesh=_MESH,
        scratch_types=[
            pltpu.VMEM((ibn, 2, CHUNK), jnp.int32),
            pltpu.VMEM((rbn, CHUNK, 128), jnp.float32),
            pltpu.VMEM((ZR, 128), jnp.float32),
            pltpu.VMEM_SHARED((NPAD, 128), jnp.float32),
            pltpu.SemaphoreType.DMA,
            pltpu.SemaphoreType.DMA,
        ],
        compiler_params=pltpu.CompilerParams(use_tc_tiling_on_sc=True),
    )
    return k(hs, ei)


def _propagate(hs, src2d, dst2d, dc):
    ibn, rbn = (4, 2) if dc == 128 else (12, 6)
    k = pl.kernel(
        _make_prop_body(ibn, rbn),
        out_type=jax.ShapeDtypeStruct((2, NPAD, dc), jnp.float32),
        mesh=_MESH,
        scratch_types=[
            pltpu.VMEM((ibn, 2, CHUNK), jnp.int32),
            pltpu.VMEM((rbn, CHUNK, dc), jnp.float32),
            pltpu.VMEM_SHARED((NPAD, dc), jnp.float32),
            pltpu.SemaphoreType.DMA,
            pltpu.SemaphoreType.DMA,
        ],
        compiler_params=pltpu.CompilerParams(use_tc_tiling_on_sc=(dc == 128)),
    )
    return k(hs, src2d, dst2d)


def _tc_first(xp, W, degp_t):
    dout = W.shape[1]

    def body(x_ref, w_ref, deg_ref, hs_ref, dis_ref):
        deg = 1.0 + deg_ref[:, 0:1] + deg_ref[:, 1:2]
        d = lax.rsqrt(deg)
        dis_ref[...] = d
        h = jnp.dot(x_ref[...], w_ref[...],
                    preferred_element_type=jnp.float32)
        hs_ref[...] = h * d

    return pl.pallas_call(
        body,
        grid=(GRID,),
        in_specs=[
            pl.BlockSpec((BLK, xp.shape[1]), lambda i: (i, 0)),
            pl.BlockSpec(W.shape, lambda i: (0, 0)),
            pl.BlockSpec((BLK, 2), lambda i: (i, 0)),
        ],
        out_specs=[
            pl.BlockSpec((BLK, dout), lambda i: (i, 0)),
            pl.BlockSpec((BLK, 1), lambda i: (i, 0)),
        ],
        out_shape=[
            jax.ShapeDtypeStruct((NPAD, dout), jnp.float32),
            jax.ShapeDtypeStruct((NPAD, 1), jnp.float32),
        ],
    )(xp, W, degp_t)


def _tc_mid(S, dis, b, W, sum_in, split_out):
    din, dout = W.shape

    def body(s_ref, dis_ref, b_ref, w_ref, out_ref):
        if sum_in:
            X = s_ref[0] + s_ref[1]
        else:
            X = jnp.concatenate([s_ref[0], s_ref[1]], axis=-1)
        d = dis_ref[...]
        X = jnp.maximum(X * d + b_ref[...], 0.0)
        h = jnp.dot(X, w_ref[...],
                    preferred_element_type=jnp.float32)
        hs = h * d
        if split_out:
            out_ref[0] = hs[:, :dout // 2]
            out_ref[1] = hs[:, dout // 2:]
        else:
            out_ref[...] = hs

    s_half = S.shape[2]
    if split_out:
        out_spec = pl.BlockSpec((2, BLK, dout // 2), lambda i: (0, i, 0))
        out_shape = jax.ShapeDtypeStruct((2, NPAD, dout // 2), jnp.float32)
    else:
        out_spec = pl.BlockSpec((BLK, dout), lambda i: (i, 0))
        out_shape = jax.ShapeDtypeStruct((NPAD, dout), jnp.float32)
    return pl.pallas_call(
        body,
        grid=(GRID,),
        in_specs=[
            pl.BlockSpec((2, BLK, s_half), lambda i: (0, i, 0)),
            pl.BlockSpec((BLK, 1), lambda i: (i, 0)),
            pl.BlockSpec((1, din), lambda i: (0, 0)),
            pl.BlockSpec(W.shape, lambda i: (0, 0)),
        ],
        out_specs=out_spec,
        out_shape=out_shape,
    )(S, dis, b, W)


def _tc_final(S, dis, b):
    dpad = 2 * S.shape[2]

    def body(s_ref, dis_ref, b_ref, out_ref):
        z = jnp.concatenate([s_ref[0], s_ref[1]], axis=-1)
        z = z * dis_ref[...] + b_ref[...]
        col = lax.broadcasted_iota(jnp.int32, z.shape, 1)
        mask = col < NUM_CLASSES
        zm = jnp.where(mask, z, -1e30)
        m = jnp.max(zm, axis=1, keepdims=True)
        e = jnp.where(mask, jnp.exp(z - m), 0.0)
        s = jnp.sum(e, axis=1, keepdims=True)
        out_ref[...] = z - (m + jnp.log(s))

    return pl.pallas_call(
        body,
        grid=(GRID,),
        in_specs=[
            pl.BlockSpec((2, BLK, dpad // 2), lambda i: (0, i, 0)),
            pl.BlockSpec((BLK, 1), lambda i: (i, 0)),
            pl.BlockSpec((1, dpad), lambda i: (0, 0)),
        ],
        out_specs=pl.BlockSpec((BLK, dpad), lambda i: (i, 0)),
        out_shape=jax.ShapeDtypeStruct((NPAD, dpad), jnp.float32),
    )(S, dis, b)


def kernel(x, edge_index, W1, b1, W2, b2, W3, b3, W4, b4):
    src2d = edge_index[0].reshape(NCHUNKS, CHUNK)
    dst2d = edge_index[1].reshape(NCHUNKS, CHUNK)
    xp = jnp.pad(x, ((0, NPAD - N), (0, 0)))
    d4p = 32
    W4p = jnp.pad(W4, ((0, 0), (0, d4p - W4.shape[1])))
    b4p = jnp.pad(b4, (0, d4p - b4.shape[0]))

    degp_t = _degrees(edge_index).T

    hs1, dis = _tc_first(xp, W1, degp_t)
    s1 = _propagate_full(hs1, edge_index)
    hs2 = _tc_mid(s1, dis, b1.reshape(1, -1), W2, True, True)
    s2 = _propagate(hs2, src2d, dst2d, W2.shape[1] // 2)
    hs3 = _tc_mid(s2, dis, b2.reshape(1, -1), W3, False, False)
    s3 = _propagate_full(hs3, edge_index)
    hs4 = _tc_mid(s3, dis, b3.reshape(1, -1), W4p, True, True)
    s4 = _propagate(hs4, src2d, dst2d, d4p // 2)
    out = _tc_final(s4, dis, b4p.reshape(1, -1))
    return out[:N, :NUM_CLASSES]
